# baseline (speedup 1.0000x reference)
import functools

import jax
import jax.numpy as jnp
from jax import lax
from jax.experimental import pallas as pl
from jax.experimental.pallas import tpu as pltpu
from jax.experimental.pallas import tpu_sc as plsc

N = 10000
E = 320000
D_NODE = 128
H = 64
L = 4

NC = 2
NS = 16
NW = NC * NS

N_PAD = 10240
E_PAD = 327680
CHUNK = 128
EPW = E_PAD // NW
NCHUNKS = EPW // CHUNK
STRIPE = N_PAD // NS
TW = 128

TE = 2048
TN = 2048

_f32 = jnp.float32


NBUF = 5
NBUF_S = 2
NSTRIPE = 2


def _make_gather_body(nch, cbase0):
    ngrp = nch // NBUF

    def body(t1_hbm, t2_hbm, rc_hbm, out_hbm, idx, buf, semi, sema, semb, semw):
        wid = lax.axis_index("s") * NC + lax.axis_index("c")
        cbase = cbase0 + wid * nch
        obase = wid * nch * CHUNK

        def stage_a(j, sl):
            pltpu.async_copy(rc_hbm.at[cbase + j], idx.at[sl], semi.at[sl])

        def stage_b(j, sl, reuse):
            pltpu.make_async_copy(rc_hbm.at[0], idx.at[sl], semi.at[sl]).wait()
            if reuse:
                pltpu.make_async_copy(buf.at[sl], out_hbm.at[pl.ds(0, CHUNK)],
                                      semw.at[sl]).wait()
            pltpu.async_copy(t1_hbm.at[idx.at[sl, 0]], buf.at[sl], sema.at[sl])

        def stage_c(j, sl):
            pltpu.make_async_copy(t1_hbm.at[idx.at[sl, 0]], buf.at[sl],
                                  sema.at[sl]).wait()
            pltpu.async_copy(t2_hbm.at[idx.at[sl, 1]], buf.at[sl], semb.at[sl],
                             add=True)

        def stage_d(j, sl):
            pltpu.make_async_copy(t2_hbm.at[idx.at[sl, 1]], buf.at[sl],
                                  semb.at[sl]).wait()
            off = obase + j * CHUNK
            pltpu.async_copy(buf.at[sl], out_hbm.at[pl.ds(off, CHUNK)],
                             semw.at[sl])

        for b in range(NBUF):
            stage_a(b, b)
            if b >= 1:
                stage_b(b - 1, b - 1, False)
            if b >= 2:
                stage_c(b - 2, b - 2)
            if b >= 3:
                stage_d(b - 3, b - 3)

        for b in range(NBUF):
            stage_a(NBUF + b, b)
            stage_b(NBUF + b - 1, (b - 1) % NBUF, b != 0)
            stage_c(NBUF + b - 2, (b - 2) % NBUF)
            stage_d(NBUF + b - 3, (b - 3) % NBUF)

        def turn(g, carry):
            j0 = g * NBUF
            for b in range(NBUF):
                stage_a(j0 + b, b)
                stage_b(j0 + b - 1, (b - 1) % NBUF, True)
                stage_c(j0 + b - 2, (b - 2) % NBUF)
                stage_d(j0 + b - 3, (b - 3) % NBUF)
            return carry

        lax.fori_loop(2, ngrp, turn, 0)

        jl = nch
        stage_b(jl - 1, (jl - 1) % NBUF, True)
        stage_c(jl - 2, (jl - 2) % NBUF)
        stage_d(jl - 3, (jl - 3) % NBUF)
        stage_c(jl - 1, (jl - 1) % NBUF)
        stage_d(jl - 2, (jl - 2) % NBUF)
        stage_d(jl - 1, (jl - 1) % NBUF)
        for sl in range(NBUF):
            pltpu.make_async_copy(buf.at[sl], out_hbm.at[pl.ds(0, CHUNK)],
                                  semw.at[sl]).wait()

    return body


def _make_scatter_body(nch, cbase0):
    ngrp = nch // NBUF_S

    def body(mw_hbm, rc_hbm, zero_hbm, out_hbm, idx, buf, acc_sh,
             semi, semd, semc):
        cid = lax.axis_index("c")
        sid = lax.axis_index("s")
        wid = sid * NC + cid
        cbase = cbase0 + wid * nch
        obase = wid * nch * CHUNK

        zoff = pl.multiple_of(sid * STRIPE, 8)
        pltpu.sync_copy(zero_hbm.at[pl.ds(zoff, STRIPE)],
                        acc_sh.at[pl.ds(zoff, STRIPE)])
        plsc.subcore_barrier()

        def fetch(j, sl):
            pltpu.async_copy(rc_hbm.at[cbase + j, 0], idx.at[sl], semi.at[sl])
            off = obase + j * CHUNK
            pltpu.async_copy(mw_hbm.at[pl.ds(off, CHUNK)], buf.at[sl],
                             semd.at[sl])

        def add(j, sl):
            pltpu.make_async_copy(rc_hbm.at[0, 0], idx.at[sl],
                                  semi.at[sl]).wait()
            pltpu.make_async_copy(mw_hbm.at[pl.ds(0, CHUNK)], buf.at[sl],
                                  semd.at[sl]).wait()
            pltpu.sync_copy(buf.at[sl], acc_sh.at[idx.at[sl]], add=True)

        for b in range(NBUF_S - 1):
            fetch(b, b)

        def turn(g, carry):
            j0 = g * NBUF_S
            for b in range(NBUF_S):
                fetch(j0 + b + NBUF_S - 1, (b + NBUF_S - 1) % NBUF_S)
                add(j0 + b, b)
            return carry

        lax.fori_loop(0, ngrp - 1, turn, 0)
        jl = (ngrp - 1) * NBUF_S
        fetch(jl + NBUF_S - 1, NBUF_S - 1)
        for b in range(NBUF_S):
            add(jl + b, b)

        plsc.subcore_barrier()
        ooff = pl.multiple_of(cid * N_PAD + sid * STRIPE, 8)
        pltpu.sync_copy(acc_sh.at[pl.ds(zoff, STRIPE)],
                        out_hbm.at[pl.ds(ooff, STRIPE)])

    return body


@functools.cache
def _sc_kernels():
    mesh = plsc.VectorSubcoreMesh(core_axis_name="c", subcore_axis_name="s")
    params = pltpu.CompilerParams(use_tc_tiling_on_sc=False)
    es = E_PAD // NSTRIPE
    nch = es // (NW * CHUNK)
    gathers, scatters = [], []
    for st in range(NSTRIPE):
        cbase0 = st * (es // CHUNK)
        gathers.append(pl.kernel(
            _make_gather_body(nch, cbase0),
            compiler_params=params,
            out_type=jax.ShapeDtypeStruct((es, TW), _f32),
            mesh=mesh,
            scratch_types=[
                pltpu.VMEM((NBUF, 2, CHUNK), jnp.int32),
                pltpu.VMEM((NBUF, CHUNK, TW), _f32),
                pltpu.SemaphoreType.DMA((NBUF,)),
                pltpu.SemaphoreType.DMA((NBUF,)),
                pltpu.SemaphoreType.DMA((NBUF,)),
                pltpu.SemaphoreType.DMA((NBUF,)),
            ],
        ))
        scatters.append(pl.kernel(
            _make_scatter_body(nch, cbase0),
            compiler_params=params,
            out_type=jax.ShapeDtypeStruct((NC * N_PAD, TW), _f32),
            mesh=mesh,
            scratch_types=[
                pltpu.VMEM((NBUF_S, CHUNK), jnp.int32),
                pltpu.VMEM((NBUF_S, CHUNK, TW), _f32),
                pltpu.VMEM_SHARED((N_PAD, TW), _f32),
                pltpu.SemaphoreType.DMA((NBUF_S,)),
                pltpu.SemaphoreType.DMA((NBUF_S,)),
                pltpu.SemaphoreType.DMA((NBUF_S,)),
            ],
        ))
    return gathers, scatters


def _gather_edges(st, t1, t2, rc):
    return _sc_kernels()[0][st](t1, t2, rc)


def _scatter_edges(st, mw, rc, zeros_tw):
    return _sc_kernels()[1][st](mw, rc, zeros_tw)


def _full(shape):
    return pl.BlockSpec(shape, lambda i: (0, 0))


def _embed_body(nf_ref, ew_ref, eb_ref, wa_ref, wb_ref,
                h_ref, cp_ref, t1_ref, t2_ref):
    nf = nf_ref[...]
    rows = pl.program_id(0) * TN + lax.broadcasted_iota(jnp.int32, (TN, 1), 0)
    valid = (rows < N).astype(_f32)
    h = (jnp.dot(nf, ew_ref[...], preferred_element_type=_f32) + eb_ref[...]) * valid
    cmask = (lax.broadcasted_iota(jnp.int32, (1, 8), 1) < 3).astype(_f32)
    cp = nf[:, :8] * cmask
    ha = jnp.dot(h, wa_ref[...], preferred_element_type=_f32)
    hb = jnp.dot(h, wb_ref[...], preferred_element_type=_f32)
    zpad = jnp.zeros((cp.shape[0], TW - H - 8), _f32)
    h_ref[...] = h
    cp_ref[...] = cp
    t1_ref[...] = jnp.concatenate([ha, cp, zpad], axis=1)
    t2_ref[...] = jnp.concatenate([hb, -cp, zpad], axis=1)


def _embed_tables(nfp, ew, eb, wa, wb):
    grid = (N_PAD // TN,)
    return pl.pallas_call(
        _embed_body,
        grid=grid,
        in_specs=[pl.BlockSpec((TN, D_NODE), lambda i: (i, 0)),
                  _full((D_NODE, H)), _full((1, H)), _full((H, H)), _full((H, H))],
        out_specs=[pl.BlockSpec((TN, H), lambda i: (i, 0)),
                   pl.BlockSpec((TN, 8), lambda i: (i, 0)),
                   pl.BlockSpec((TN, TW), lambda i: (i, 0)),
                   pl.BlockSpec((TN, TW), lambda i: (i, 0))],
        out_shape=[jax.ShapeDtypeStruct((N_PAD, H), _f32),
                   jax.ShapeDtypeStruct((N_PAD, 8), _f32),
                   jax.ShapeDtypeStruct((N_PAD, TW), _f32),
                   jax.ShapeDtypeStruct((N_PAD, TW), _f32)],
    )(nfp, ew, eb, wa, wb)


def _edge_body(g_ref, ea_ref, wc_ref, wd_ref, e1b_ref, e2w_ref, e2b_ref,
               c1w_ref, c1b_ref, c2w_ref, c2b_ref, out_ref):
    g = g_ref[...]
    gh = g[:, :H]
    diff = g[:, H:H + 8]
    dist2 = jnp.sum(diff * diff, axis=1, keepdims=True)
    pre = (gh + dist2 * wc_ref[...] + e1b_ref[...]
           + jnp.dot(ea_ref[...], wd_ref[...], preferred_element_type=_f32))
    m1 = jax.nn.silu(pre)
    m = jax.nn.silu(jnp.dot(m1, e2w_ref[...], preferred_element_type=_f32)
                    + e2b_ref[...])
    cm1 = jax.nn.silu(jnp.dot(m, c1w_ref[...], preferred_element_type=_f32)
                      + c1b_ref[...])
    cm = jnp.dot(cm1, c2w_ref[...], preferred_element_type=_f32) + c2b_ref[...]
    wdiff = diff * cm
    zpad = jnp.zeros((m.shape[0], TW - H - 8), _f32)
    out_ref[...] = jnp.concatenate([m, wdiff, zpad], axis=1)


def _edge_mlp(g, eap, wc, wd, e1b, e2w, e2b, c1w, c1b, c2w, c2b):
    es = E_PAD // NSTRIPE
    grid = (es // TE,)
    return pl.pallas_call(
        _edge_body,
        grid=grid,
        in_specs=[pl.BlockSpec((TE, TW), lambda i: (i, 0)),
                  pl.BlockSpec((TE, 8), lambda i: (i, 0)),
                  _full((1, H)), _full((8, H)), _full((1, H)),
                  _full((H, H)), _full((1, H)),
                  _full((H, H)), _full((1, H)),
                  _full((H, 8)), _full((1, 8))],
        out_specs=pl.BlockSpec((TE, TW), lambda i: (i, 0)),
        out_shape=jax.ShapeDtypeStruct((E_PAD // NSTRIPE, TW), _f32),
    )(g, eap, wc, wd, e1b, e2w, e2b, c1w, c1b, c2w, c2b)


def _node_body(h_ref, cp_ref, p_ref,
               n1h_ref, n1a_ref, n1b_ref,
               n2w_ref, n2b_ref, wa_ref, wb_ref,
               hn_ref, cpn_ref, t1_ref, t2_ref):
    h = h_ref[...]
    s = jnp.sum(p_ref[...], axis=0)
    agg = s[:, :H]
    rows = pl.program_id(0) * TN + lax.broadcasted_iota(jnp.int32, (TN, 1), 0)
    valid = (rows < N).astype(_f32)
    cpn = (cp_ref[...] + s[:, H:H + 8]) * valid
    t = jnp.maximum(
        jnp.dot(h, n1h_ref[...], preferred_element_type=_f32)
        + jnp.dot(agg, n1a_ref[...], preferred_element_type=_f32)
        + n1b_ref[...], 0.0)
    hn = (h + jnp.dot(t, n2w_ref[...], preferred_element_type=_f32)
          + n2b_ref[...]) * valid
    ha = jnp.dot(hn, wa_ref[...], preferred_element_type=_f32)
    hb = jnp.dot(hn, wb_ref[...], preferred_element_type=_f32)
    zpad = jnp.zeros((cpn.shape[0], TW - H - 8), _f32)
    hn_ref[...] = hn
    cpn_ref[...] = cpn
    t1_ref[...] = jnp.concatenate([ha, cpn, zpad], axis=1)
    t2_ref[...] = jnp.concatenate([hb, -cpn, zpad], axis=1)


def _node_update(h, cp, pstack, n1h, n1a, n1b, n2w, n2b, wa, wb):
    grid = (N_PAD // TN,)
    return pl.pallas_call(
        _node_body,
        grid=grid,
        in_specs=[pl.BlockSpec((TN, H), lambda i: (i, 0)),
                  pl.BlockSpec((TN, 8), lambda i: (i, 0)),
                  pl.BlockSpec((2 * NSTRIPE, TN, TW), lambda i: (0, i, 0)),
                  _full((H, H)), _full((H, H)), _full((1, H)),
                  _full((H, H)), _full((1, H)),
                  _full((H, H)), _full((H, H))],
        out_specs=[pl.BlockSpec((TN, H), lambda i: (i, 0)),
                   pl.BlockSpec((TN, 8), lambda i: (i, 0)),
                   pl.BlockSpec((TN, TW), lambda i: (i, 0)),
                   pl.BlockSpec((TN, TW), lambda i: (i, 0))],
        out_shape=[jax.ShapeDtypeStruct((N_PAD, H), _f32),
                   jax.ShapeDtypeStruct((N_PAD, 8), _f32),
                   jax.ShapeDtypeStruct((N_PAD, TW), _f32),
                   jax.ShapeDtypeStruct((N_PAD, TW), _f32)],
    )(h, cp, pstack, n1h, n1a, n1b, n2w, n2b, wa, wb)


def _split_edge_weights(lp):
    e1 = lp["e1_w"]
    wa = e1[:H]
    wb = e1[H:2 * H]
    wc = e1[2 * H:2 * H + 1]
    wd = jnp.zeros((8, H), _f32).at[:4].set(e1[2 * H + 1:])
    c2w = jnp.zeros((H, 8), _f32).at[:, :3].set(lp["c2_w"])
    c2b = jnp.zeros((1, 8), _f32).at[0, :3].set(lp["c2_b"])
    return wa, wb, wc, wd, c2w, c2b


def kernel(node_features, edge_indices, edges_features, batch_size, params):
    row = edge_indices[0].astype(jnp.int32)
    col = edge_indices[1].astype(jnp.int32)
    pad = jnp.full((E_PAD - E,), N, jnp.int32)
    rowp = jnp.concatenate([row, pad])
    colp = jnp.concatenate([col, pad])
    rc = jnp.stack([rowp.reshape(-1, CHUNK), colp.reshape(-1, CHUNK)], axis=1)
    eap_full = jnp.zeros((E_PAD, 8), _f32).at[:E, :4].set(edges_features)
    es = E_PAD // NSTRIPE
    eap = [eap_full[st * es:(st + 1) * es] for st in range(NSTRIPE)]
    nfp = jnp.zeros((N_PAD, D_NODE), _f32).at[:N].set(node_features)
    zeros_tw = jnp.zeros((N_PAD, TW), _f32)

    ew = jnp.concatenate([jnp.zeros((3, H), _f32), params["emb_w"]])
    eb = params["emb_b"].reshape(1, H)

    layers = params["layers"]
    split = [_split_edge_weights(lp) for lp in layers]

    wa0, wb0 = split[0][0], split[0][1]
    h, cp, t1, t2 = _embed_tables(nfp, ew, eb, wa0, wb0)

    zh = jnp.zeros((H, H), _f32)
    for l in range(L):
        lp = layers[l]
        _, _, wc, wd, c2w, c2b = split[l]
        parts = []
        for st in range(NSTRIPE):
            g = _gather_edges(st, t1, t2, rc)
            mw = _edge_mlp(g, eap[st], wc, wd,
                           lp["e1_b"].reshape(1, H), lp["e2_w"],
                           lp["e2_b"].reshape(1, H), lp["c1_w"],
                           lp["c1_b"].reshape(1, H), c2w, c2b)
            parts.append(_scatter_edges(st, mw, rc, zeros_tw))
        wan, wbn = (split[l + 1][0], split[l + 1][1]) if l + 1 < L else (zh, zh)
        pstack = jnp.concatenate(
            [pt.reshape(2, N_PAD, TW) for pt in parts], axis=0)
        h, cp, t1, t2 = _node_update(
            h, cp, pstack,
            lp["n1_w"][:H], lp["n1_w"][H:], lp["n1_b"].reshape(1, H),
            lp["n2_w"], lp["n2_b"].reshape(1, H), wan, wbn)

    pred = cp[:N, :3]
    return (pred, edge_indices, edges_features)


# speedup vs baseline: 1.1178x; 1.1178x over previous
import functools

import jax
import jax.numpy as jnp
from jax import lax
from jax.experimental import pallas as pl
from jax.experimental.pallas import tpu as pltpu
from jax.experimental.pallas import tpu_sc as plsc

N = 10000
E = 320000
D_NODE = 128
H = 64
L = 4

NC = 2
NS = 16
NW = NC * NS

N_PAD = 10240
E_PAD = 327680
CHUNK = 128
EPW = E_PAD // NW
NCHUNKS = EPW // CHUNK
STRIPE = N_PAD // NS
TW = 128

TE = 2048
TN = 2048

_f32 = jnp.float32


NBUF = 4
NBUF_S = 2
NSTRIPE = 2


def _make_gather_body(nch, cbase0):
    ngrp = nch // NBUF

    def body(t1_hbm, t2_hbm, rc_hbm, out_hbm, idx, buf, semi, sema, semb, semw):
        wid = lax.axis_index("s") * NC + lax.axis_index("c")
        cbase = cbase0 + wid * nch
        obase = wid * nch * CHUNK

        def stage_a(j, sl):
            pltpu.async_copy(rc_hbm.at[cbase + j], idx.at[sl], semi.at[sl])

        def stage_b(j, sl, reuse):
            pltpu.make_async_copy(rc_hbm.at[0], idx.at[sl], semi.at[sl]).wait()
            if reuse:
                pltpu.make_async_copy(buf.at[sl], out_hbm.at[pl.ds(0, CHUNK)],
                                      semw.at[sl]).wait()
            pltpu.async_copy(t1_hbm.at[idx.at[sl, 0]], buf.at[sl], sema.at[sl])

        def stage_c(j, sl):
            pltpu.make_async_copy(t1_hbm.at[idx.at[sl, 0]], buf.at[sl],
                                  sema.at[sl]).wait()
            pltpu.async_copy(t2_hbm.at[idx.at[sl, 1]], buf.at[sl], semb.at[sl],
                             add=True)

        def stage_d(j, sl):
            pltpu.make_async_copy(t2_hbm.at[idx.at[sl, 1]], buf.at[sl],
                                  semb.at[sl]).wait()
            off = obase + j * CHUNK
            pltpu.async_copy(buf.at[sl], out_hbm.at[pl.ds(off, CHUNK)],
                             semw.at[sl])

        for b in range(NBUF):
            stage_a(b, b)
            if b >= 1:
                stage_b(b - 1, b - 1, False)
            if b >= 2:
                stage_c(b - 2, b - 2)
            if b >= 3:
                stage_d(b - 3, b - 3)

        for b in range(NBUF):
            stage_a(NBUF + b, b)
            stage_b(NBUF + b - 1, (b - 1) % NBUF, b != 0)
            stage_c(NBUF + b - 2, (b - 2) % NBUF)
            stage_d(NBUF + b - 3, (b - 3) % NBUF)

        def turn(g, carry):
            j0 = g * NBUF
            for b in range(NBUF):
                stage_a(j0 + b, b)
                stage_b(j0 + b - 1, (b - 1) % NBUF, True)
                stage_c(j0 + b - 2, (b - 2) % NBUF)
                stage_d(j0 + b - 3, (b - 3) % NBUF)
            return carry

        lax.fori_loop(2, ngrp, turn, 0)

        jl = nch
        stage_b(jl - 1, (jl - 1) % NBUF, True)
        stage_c(jl - 2, (jl - 2) % NBUF)
        stage_d(jl - 3, (jl - 3) % NBUF)
        stage_c(jl - 1, (jl - 1) % NBUF)
        stage_d(jl - 2, (jl - 2) % NBUF)
        stage_d(jl - 1, (jl - 1) % NBUF)
        for sl in range(NBUF):
            pltpu.make_async_copy(buf.at[sl], out_hbm.at[pl.ds(0, CHUNK)],
                                  semw.at[sl]).wait()

    return body


def _make_scatter_body(nch, cbase0):
    ngrp = nch // NBUF_S

    def body(mw_hbm, rc_hbm, zero_hbm, out_hbm, idx, buf, acc_sh,
             semi, semd, semc):
        cid = lax.axis_index("c")
        sid = lax.axis_index("s")
        wid = sid * NC + cid
        cbase = cbase0 + wid * nch
        obase = wid * nch * CHUNK

        zoff = pl.multiple_of(sid * STRIPE, 8)
        pltpu.sync_copy(zero_hbm.at[pl.ds(zoff, STRIPE)],
                        acc_sh.at[pl.ds(zoff, STRIPE)])
        plsc.subcore_barrier()

        def fetch(j, sl):
            pltpu.async_copy(rc_hbm.at[cbase + j, 0], idx.at[sl], semi.at[sl])
            off = obase + j * CHUNK
            pltpu.async_copy(mw_hbm.at[pl.ds(off, CHUNK)], buf.at[sl],
                             semd.at[sl])

        def add(j, sl):
            pltpu.make_async_copy(rc_hbm.at[0, 0], idx.at[sl],
                                  semi.at[sl]).wait()
            pltpu.make_async_copy(mw_hbm.at[pl.ds(0, CHUNK)], buf.at[sl],
                                  semd.at[sl]).wait()
            pltpu.sync_copy(buf.at[sl], acc_sh.at[idx.at[sl]], add=True)

        for b in range(NBUF_S - 1):
            fetch(b, b)

        def turn(g, carry):
            j0 = g * NBUF_S
            for b in range(NBUF_S):
                fetch(j0 + b + NBUF_S - 1, (b + NBUF_S - 1) % NBUF_S)
                add(j0 + b, b)
            return carry

        lax.fori_loop(0, ngrp - 1, turn, 0)
        jl = (ngrp - 1) * NBUF_S
        fetch(jl + NBUF_S - 1, NBUF_S - 1)
        for b in range(NBUF_S):
            add(jl + b, b)

        plsc.subcore_barrier()
        ooff = pl.multiple_of(cid * N_PAD + sid * STRIPE, 8)
        pltpu.sync_copy(acc_sh.at[pl.ds(zoff, STRIPE)],
                        out_hbm.at[pl.ds(ooff, STRIPE)])

    return body


@functools.cache
def _sc_kernels():
    mesh = plsc.VectorSubcoreMesh(core_axis_name="c", subcore_axis_name="s")
    params = pltpu.CompilerParams(use_tc_tiling_on_sc=False)
    es = E_PAD // NSTRIPE
    nch = es // (NW * CHUNK)
    gathers, scatters = [], []
    for st in range(NSTRIPE):
        cbase0 = st * (es // CHUNK)
        gathers.append(pl.kernel(
            _make_gather_body(nch, cbase0),
            compiler_params=params,
            out_type=jax.ShapeDtypeStruct((es, TW), _f32),
            mesh=mesh,
            scratch_types=[
                pltpu.VMEM((NBUF, 2, CHUNK), jnp.int32),
                pltpu.VMEM((NBUF, CHUNK, TW), _f32),
                pltpu.SemaphoreType.DMA((NBUF,)),
                pltpu.SemaphoreType.DMA((NBUF,)),
                pltpu.SemaphoreType.DMA((NBUF,)),
                pltpu.SemaphoreType.DMA((NBUF,)),
            ],
        ))
        scatters.append(pl.kernel(
            _make_scatter_body(nch, cbase0),
            compiler_params=params,
            out_type=jax.ShapeDtypeStruct((NC * N_PAD, TW), _f32),
            mesh=mesh,
            scratch_types=[
                pltpu.VMEM((NBUF_S, CHUNK), jnp.int32),
                pltpu.VMEM((NBUF_S, CHUNK, TW), _f32),
                pltpu.VMEM_SHARED((N_PAD, TW), _f32),
                pltpu.SemaphoreType.DMA((NBUF_S,)),
                pltpu.SemaphoreType.DMA((NBUF_S,)),
                pltpu.SemaphoreType.DMA((NBUF_S,)),
            ],
        ))
    return gathers, scatters


def _gather_edges(st, t1, t2, rc):
    return _sc_kernels()[0][st](t1, t2, rc)


def _scatter_edges(st, mw, rc, zeros_tw):
    return _sc_kernels()[1][st](mw, rc, zeros_tw)


def _full(shape):
    return pl.BlockSpec(shape, lambda i: (0, 0))


def _embed_body(nf_ref, ew_ref, eb_ref, wa_ref, wb_ref,
                h_ref, cp_ref, t1_ref, t2_ref):
    nf = nf_ref[...]
    rows = pl.program_id(0) * TN + lax.broadcasted_iota(jnp.int32, (TN, 1), 0)
    valid = (rows < N).astype(_f32)
    h = (jnp.dot(nf, ew_ref[...], preferred_element_type=_f32) + eb_ref[...]) * valid
    cmask = (lax.broadcasted_iota(jnp.int32, (1, 8), 1) < 3).astype(_f32)
    cp = nf[:, :8] * cmask
    ha = jnp.dot(h, wa_ref[...], preferred_element_type=_f32)
    hb = jnp.dot(h, wb_ref[...], preferred_element_type=_f32)
    zpad = jnp.zeros((cp.shape[0], TW - H - 8), _f32)
    h_ref[...] = h
    cp_ref[...] = cp
    t1_ref[...] = jnp.concatenate([ha, cp, zpad], axis=1)
    t2_ref[...] = jnp.concatenate([hb, -cp, zpad], axis=1)


def _embed_tables(nfp, ew, eb, wa, wb):
    grid = (N_PAD // TN,)
    return pl.pallas_call(
        _embed_body,
        grid=grid,
        in_specs=[pl.BlockSpec((TN, D_NODE), lambda i: (i, 0)),
                  _full((D_NODE, H)), _full((1, H)), _full((H, H)), _full((H, H))],
        out_specs=[pl.BlockSpec((TN, H), lambda i: (i, 0)),
                   pl.BlockSpec((TN, 8), lambda i: (i, 0)),
                   pl.BlockSpec((TN, TW), lambda i: (i, 0)),
                   pl.BlockSpec((TN, TW), lambda i: (i, 0))],
        out_shape=[jax.ShapeDtypeStruct((N_PAD, H), _f32),
                   jax.ShapeDtypeStruct((N_PAD, 8), _f32),
                   jax.ShapeDtypeStruct((N_PAD, TW), _f32),
                   jax.ShapeDtypeStruct((N_PAD, TW), _f32)],
    )(nfp, ew, eb, wa, wb)


def _edge_body(g_ref, ea_ref, wc_ref, wd_ref, e1b_ref, e2w_ref, e2b_ref,
               c1w_ref, c1b_ref, c2w_ref, c2b_ref, out_ref):
    g = g_ref[...]
    gh = g[:, :H]
    diff = g[:, H:H + 8]
    dist2 = jnp.sum(diff * diff, axis=1, keepdims=True)
    pre = (gh + dist2 * wc_ref[...] + e1b_ref[...]
           + jnp.dot(ea_ref[...], wd_ref[...], preferred_element_type=_f32))
    m1 = jax.nn.silu(pre)
    m = jax.nn.silu(jnp.dot(m1, e2w_ref[...], preferred_element_type=_f32)
                    + e2b_ref[...])
    cm1 = jax.nn.silu(jnp.dot(m, c1w_ref[...], preferred_element_type=_f32)
                      + c1b_ref[...])
    cm = jnp.dot(cm1, c2w_ref[...], preferred_element_type=_f32) + c2b_ref[...]
    wdiff = diff * cm
    zpad = jnp.zeros((m.shape[0], TW - H - 8), _f32)
    out_ref[...] = jnp.concatenate([m, wdiff, zpad], axis=1)


def _edge_mlp(g, eap, wc, wd, e1b, e2w, e2b, c1w, c1b, c2w, c2b):
    es = E_PAD // NSTRIPE
    grid = (es // TE,)
    return pl.pallas_call(
        _edge_body,
        grid=grid,
        in_specs=[pl.BlockSpec((TE, TW), lambda i: (i, 0)),
                  pl.BlockSpec((TE, 8), lambda i: (i, 0)),
                  _full((1, H)), _full((8, H)), _full((1, H)),
                  _full((H, H)), _full((1, H)),
                  _full((H, H)), _full((1, H)),
                  _full((H, 8)), _full((1, 8))],
        out_specs=pl.BlockSpec((TE, TW), lambda i: (i, 0)),
        out_shape=jax.ShapeDtypeStruct((E_PAD // NSTRIPE, TW), _f32),
    )(g, eap, wc, wd, e1b, e2w, e2b, c1w, c1b, c2w, c2b)


def _node_body(h_ref, cp_ref, p0_ref, p1_ref, p2_ref, p3_ref,
               n1h_ref, n1a_ref, n1b_ref,
               n2w_ref, n2b_ref, wa_ref, wb_ref,
               hn_ref, cpn_ref, t1_ref, t2_ref):
    h = h_ref[...]
    s = (p0_ref[...] + p1_ref[...]) + (p2_ref[...] + p3_ref[...])
    agg = s[:, :H]
    rows = pl.program_id(0) * TN + lax.broadcasted_iota(jnp.int32, (TN, 1), 0)
    valid = (rows < N).astype(_f32)
    cpn = (cp_ref[...] + s[:, H:H + 8]) * valid
    t = jnp.maximum(
        jnp.dot(h, n1h_ref[...], preferred_element_type=_f32)
        + jnp.dot(agg, n1a_ref[...], preferred_element_type=_f32)
        + n1b_ref[...], 0.0)
    hn = (h + jnp.dot(t, n2w_ref[...], preferred_element_type=_f32)
          + n2b_ref[...]) * valid
    ha = jnp.dot(hn, wa_ref[...], preferred_element_type=_f32)
    hb = jnp.dot(hn, wb_ref[...], preferred_element_type=_f32)
    zpad = jnp.zeros((cpn.shape[0], TW - H - 8), _f32)
    hn_ref[...] = hn
    cpn_ref[...] = cpn
    t1_ref[...] = jnp.concatenate([ha, cpn, zpad], axis=1)
    t2_ref[...] = jnp.concatenate([hb, -cpn, zpad], axis=1)


def _node_update(h, cp, p0, p1, p2, p3, n1h, n1a, n1b, n2w, n2b, wa, wb):
    grid = (N_PAD // TN,)
    return pl.pallas_call(
        _node_body,
        grid=grid,
        in_specs=[pl.BlockSpec((TN, H), lambda i: (i, 0)),
                  pl.BlockSpec((TN, 8), lambda i: (i, 0)),
                  pl.BlockSpec((TN, TW), lambda i: (i, 0)),
                  pl.BlockSpec((TN, TW), lambda i: (i, 0)),
                  pl.BlockSpec((TN, TW), lambda i: (i, 0)),
                  pl.BlockSpec((TN, TW), lambda i: (i, 0)),
                  _full((H, H)), _full((H, H)), _full((1, H)),
                  _full((H, H)), _full((1, H)),
                  _full((H, H)), _full((H, H))],
        out_specs=[pl.BlockSpec((TN, H), lambda i: (i, 0)),
                   pl.BlockSpec((TN, 8), lambda i: (i, 0)),
                   pl.BlockSpec((TN, TW), lambda i: (i, 0)),
                   pl.BlockSpec((TN, TW), lambda i: (i, 0))],
        out_shape=[jax.ShapeDtypeStruct((N_PAD, H), _f32),
                   jax.ShapeDtypeStruct((N_PAD, 8), _f32),
                   jax.ShapeDtypeStruct((N_PAD, TW), _f32),
                   jax.ShapeDtypeStruct((N_PAD, TW), _f32)],
    )(h, cp, p0, p1, p2, p3, n1h, n1a, n1b, n2w, n2b, wa, wb)


def _split_edge_weights(lp):
    e1 = lp["e1_w"]
    wa = e1[:H]
    wb = e1[H:2 * H]
    wc = e1[2 * H:2 * H + 1]
    wd = jnp.zeros((8, H), _f32).at[:4].set(e1[2 * H + 1:])
    c2w = jnp.zeros((H, 8), _f32).at[:, :3].set(lp["c2_w"])
    c2b = jnp.zeros((1, 8), _f32).at[0, :3].set(lp["c2_b"])
    return wa, wb, wc, wd, c2w, c2b


def kernel(node_features, edge_indices, edges_features, batch_size, params):
    row = edge_indices[0].astype(jnp.int32)
    col = edge_indices[1].astype(jnp.int32)
    pad = jnp.full((E_PAD - E,), N, jnp.int32)
    rowp = jnp.concatenate([row, pad])
    colp = jnp.concatenate([col, pad])
    rc = jnp.stack([rowp.reshape(-1, CHUNK), colp.reshape(-1, CHUNK)], axis=1)
    eap_full = jnp.zeros((E_PAD, 8), _f32).at[:E, :4].set(edges_features)
    es = E_PAD // NSTRIPE
    eap = [eap_full[st * es:(st + 1) * es] for st in range(NSTRIPE)]
    nfp = jnp.zeros((N_PAD, D_NODE), _f32).at[:N].set(node_features)
    zeros_tw = jnp.zeros((N_PAD, TW), _f32)

    ew = jnp.concatenate([jnp.zeros((3, H), _f32), params["emb_w"]])
    eb = params["emb_b"].reshape(1, H)

    layers = params["layers"]
    split = [_split_edge_weights(lp) for lp in layers]

    wa0, wb0 = split[0][0], split[0][1]
    h, cp, t1, t2 = _embed_tables(nfp, ew, eb, wa0, wb0)

    zh = jnp.zeros((H, H), _f32)
    for l in range(L):
        lp = layers[l]
        _, _, wc, wd, c2w, c2b = split[l]
        parts = []
        for st in range(NSTRIPE):
            g = _gather_edges(st, t1, t2, rc)
            mw = _edge_mlp(g, eap[st], wc, wd,
                           lp["e1_b"].reshape(1, H), lp["e2_w"],
                           lp["e2_b"].reshape(1, H), lp["c1_w"],
                           lp["c1_b"].reshape(1, H), c2w, c2b)
            parts.append(_scatter_edges(st, mw, rc, zeros_tw))
        wan, wbn = (split[l + 1][0], split[l + 1][1]) if l + 1 < L else (zh, zh)
        h, cp, t1, t2 = _node_update(
            h, cp, parts[0][:N_PAD], parts[0][N_PAD:],
            parts[1][:N_PAD], parts[1][N_PAD:],
            lp["n1_w"][:H], lp["n1_w"][H:], lp["n1_b"].reshape(1, H),
            lp["n2_w"], lp["n2_b"].reshape(1, H), wan, wbn)

    pred = cp[:N, :3]
    return (pred, edge_indices, edges_features)


# speedup vs baseline: 1.1190x; 1.0011x over previous
import functools

import jax
import jax.numpy as jnp
from jax import lax
from jax.experimental import pallas as pl
from jax.experimental.pallas import tpu as pltpu
from jax.experimental.pallas import tpu_sc as plsc

N = 10000
E = 320000
D_NODE = 128
H = 64
L = 4

NC = 2
NS = 16
NW = NC * NS

N_PAD = 10240
E_PAD = 327680
CHUNK = 128
EPW = E_PAD // NW
NCHUNKS = EPW // CHUNK
STRIPE = N_PAD // NS
TW = 128

TE = 2048
TN = 2048

_f32 = jnp.float32


NBUF = 4
NBUF_S = 2
NSTRIPE = 2


def _make_gather_body(nch, cbase0):
    ngrp = nch // NBUF

    def body(t1_hbm, t2_hbm, rc_hbm, out_hbm, idx, buf, semi, sema, semb, semw):
        wid = lax.axis_index("s") * NC + lax.axis_index("c")
        cbase = cbase0 + wid * nch
        obase = wid * nch * CHUNK

        def stage_a(j, sl):
            pltpu.async_copy(rc_hbm.at[cbase + j], idx.at[sl], semi.at[sl])

        def stage_b(j, sl, reuse):
            pltpu.make_async_copy(rc_hbm.at[0], idx.at[sl], semi.at[sl]).wait()
            if reuse:
                pltpu.make_async_copy(buf.at[sl], out_hbm.at[pl.ds(0, CHUNK)],
                                      semw.at[sl]).wait()
            pltpu.async_copy(t1_hbm.at[idx.at[sl, 0]], buf.at[sl], sema.at[sl])

        def stage_c(j, sl):
            pltpu.make_async_copy(t1_hbm.at[idx.at[sl, 0]], buf.at[sl],
                                  sema.at[sl]).wait()
            pltpu.async_copy(t2_hbm.at[idx.at[sl, 1]], buf.at[sl], semb.at[sl],
                             add=True)

        def stage_d(j, sl):
            pltpu.make_async_copy(t2_hbm.at[idx.at[sl, 1]], buf.at[sl],
                                  semb.at[sl]).wait()
            off = obase + j * CHUNK
            pltpu.async_copy(buf.at[sl], out_hbm.at[pl.ds(off, CHUNK)],
                             semw.at[sl])

        for b in range(NBUF):
            stage_a(b, b)
            if b >= 1:
                stage_b(b - 1, b - 1, False)
            if b >= 2:
                stage_c(b - 2, b - 2)
            if b >= 3:
                stage_d(b - 3, b - 3)

        for b in range(NBUF):
            stage_a(NBUF + b, b)
            stage_b(NBUF + b - 1, (b - 1) % NBUF, b != 0)
            stage_c(NBUF + b - 2, (b - 2) % NBUF)
            stage_d(NBUF + b - 3, (b - 3) % NBUF)

        def turn(g, carry):
            j0 = g * NBUF
            for b in range(NBUF):
                stage_a(j0 + b, b)
                stage_b(j0 + b - 1, (b - 1) % NBUF, True)
                stage_c(j0 + b - 2, (b - 2) % NBUF)
                stage_d(j0 + b - 3, (b - 3) % NBUF)
            return carry

        lax.fori_loop(2, ngrp, turn, 0)

        jl = nch
        stage_b(jl - 1, (jl - 1) % NBUF, True)
        stage_c(jl - 2, (jl - 2) % NBUF)
        stage_d(jl - 3, (jl - 3) % NBUF)
        stage_c(jl - 1, (jl - 1) % NBUF)
        stage_d(jl - 2, (jl - 2) % NBUF)
        stage_d(jl - 1, (jl - 1) % NBUF)
        for sl in range(NBUF):
            pltpu.make_async_copy(buf.at[sl], out_hbm.at[pl.ds(0, CHUNK)],
                                  semw.at[sl]).wait()

    return body


def _make_scatter_body(nch, cbase0):
    ngrp = nch // NBUF_S

    def body(mw_hbm, rc_hbm, zero_hbm, out_hbm, idx, buf, acc_sh,
             semi, semd, semc):
        cid = lax.axis_index("c")
        sid = lax.axis_index("s")
        wid = sid * NC + cid
        cbase = cbase0 + wid * nch
        obase = wid * nch * CHUNK

        zoff = pl.multiple_of(sid * STRIPE, 8)
        pltpu.sync_copy(zero_hbm.at[pl.ds(zoff, STRIPE)],
                        acc_sh.at[pl.ds(zoff, STRIPE)])
        plsc.subcore_barrier()

        def fetch(j, sl):
            pltpu.async_copy(rc_hbm.at[cbase + j, 0], idx.at[sl], semi.at[sl])
            off = obase + j * CHUNK
            pltpu.async_copy(mw_hbm.at[pl.ds(off, CHUNK)], buf.at[sl],
                             semd.at[sl])

        def add(j, sl):
            pltpu.make_async_copy(rc_hbm.at[0, 0], idx.at[sl],
                                  semi.at[sl]).wait()
            pltpu.make_async_copy(mw_hbm.at[pl.ds(0, CHUNK)], buf.at[sl],
                                  semd.at[sl]).wait()
            pltpu.sync_copy(buf.at[sl], acc_sh.at[idx.at[sl]], add=True)

        for b in range(NBUF_S - 1):
            fetch(b, b)

        def turn(g, carry):
            j0 = g * NBUF_S
            for b in range(NBUF_S):
                fetch(j0 + b + NBUF_S - 1, (b + NBUF_S - 1) % NBUF_S)
                add(j0 + b, b)
            return carry

        lax.fori_loop(0, ngrp - 1, turn, 0)
        jl = (ngrp - 1) * NBUF_S
        fetch(jl + NBUF_S - 1, NBUF_S - 1)
        for b in range(NBUF_S):
            add(jl + b, b)

        plsc.subcore_barrier()
        ooff = pl.multiple_of(cid * N_PAD + sid * STRIPE, 8)
        pltpu.sync_copy(acc_sh.at[pl.ds(zoff, STRIPE)],
                        out_hbm.at[pl.ds(ooff, STRIPE)])

    return body


@functools.cache
def _sc_kernels():
    mesh = plsc.VectorSubcoreMesh(core_axis_name="c", subcore_axis_name="s")
    params = pltpu.CompilerParams(use_tc_tiling_on_sc=False)
    es = E_PAD // NSTRIPE
    nch = es // (NW * CHUNK)
    gathers, scatters = [], []
    for st in range(NSTRIPE):
        cbase0 = st * (es // CHUNK)
        gathers.append(pl.kernel(
            _make_gather_body(nch, cbase0),
            compiler_params=params,
            out_type=jax.ShapeDtypeStruct((es, TW), _f32),
            mesh=mesh,
            scratch_types=[
                pltpu.VMEM((NBUF, 2, CHUNK), jnp.int32),
                pltpu.VMEM((NBUF, CHUNK, TW), _f32),
                pltpu.SemaphoreType.DMA((NBUF,)),
                pltpu.SemaphoreType.DMA((NBUF,)),
                pltpu.SemaphoreType.DMA((NBUF,)),
                pltpu.SemaphoreType.DMA((NBUF,)),
            ],
        ))
        scatters.append(pl.kernel(
            _make_scatter_body(nch, cbase0),
            compiler_params=params,
            out_type=jax.ShapeDtypeStruct((NC * N_PAD, TW), _f32),
            mesh=mesh,
            scratch_types=[
                pltpu.VMEM((NBUF_S, CHUNK), jnp.int32),
                pltpu.VMEM((NBUF_S, CHUNK, TW), _f32),
                pltpu.VMEM_SHARED((N_PAD, TW), _f32),
                pltpu.SemaphoreType.DMA((NBUF_S,)),
                pltpu.SemaphoreType.DMA((NBUF_S,)),
                pltpu.SemaphoreType.DMA((NBUF_S,)),
            ],
        ))
    return gathers, scatters


def _gather_edges(st, t1, t2, rc):
    return _sc_kernels()[0][st](t1, t2, rc)


def _scatter_edges(st, mw, rc, zeros_tw):
    return _sc_kernels()[1][st](mw, rc, zeros_tw)


def _full(shape):
    return pl.BlockSpec(shape, lambda i: (0, 0))


def _embed_body(nf_ref, ew_ref, eb_ref, wa_ref, wb_ref,
                h_ref, cp_ref, t1_ref, t2_ref):
    nf = nf_ref[...]
    rows = pl.program_id(0) * TN + lax.broadcasted_iota(jnp.int32, (TN, 1), 0)
    valid = (rows < N).astype(_f32)
    h = (jnp.dot(nf, ew_ref[...], preferred_element_type=_f32) + eb_ref[...]) * valid
    cmask = (lax.broadcasted_iota(jnp.int32, (1, 8), 1) < 3).astype(_f32)
    cp = nf[:, :8] * cmask
    ha = jnp.dot(h, wa_ref[...], preferred_element_type=_f32)
    hb = jnp.dot(h, wb_ref[...], preferred_element_type=_f32)
    zpad = jnp.zeros((cp.shape[0], TW - H - 8), _f32)
    h_ref[...] = h
    cp_ref[...] = cp
    t1_ref[...] = jnp.concatenate([ha, cp, zpad], axis=1)
    t2_ref[...] = jnp.concatenate([hb, -cp, zpad], axis=1)


def _embed_tables(nfp, ew, eb, wa, wb):
    grid = (N_PAD // TN,)
    return pl.pallas_call(
        _embed_body,
        grid=grid,
        in_specs=[pl.BlockSpec((TN, D_NODE), lambda i: (i, 0)),
                  _full((D_NODE, H)), _full((1, H)), _full((H, H)), _full((H, H))],
        out_specs=[pl.BlockSpec((TN, H), lambda i: (i, 0)),
                   pl.BlockSpec((TN, 8), lambda i: (i, 0)),
                   pl.BlockSpec((TN, TW), lambda i: (i, 0)),
                   pl.BlockSpec((TN, TW), lambda i: (i, 0))],
        out_shape=[jax.ShapeDtypeStruct((N_PAD, H), _f32),
                   jax.ShapeDtypeStruct((N_PAD, 8), _f32),
                   jax.ShapeDtypeStruct((N_PAD, TW), _f32),
                   jax.ShapeDtypeStruct((N_PAD, TW), _f32)],
    )(nfp, ew, eb, wa, wb)


def _edge_body(g_ref, ea_ref, wc_ref, wd_ref, e1b_ref, e2w_ref, e2b_ref,
               c1w_ref, c1b_ref, c2w_ref, c2b_ref, out_ref):
    g = g_ref[...]
    gh = g[:, :H]
    diff = g[:, H:H + 8]
    dist2 = jnp.sum(diff * diff, axis=1, keepdims=True)
    pre = (gh + dist2 * wc_ref[...] + e1b_ref[...]
           + jnp.dot(ea_ref[...], wd_ref[...], preferred_element_type=_f32))
    m1 = jax.nn.silu(pre)
    m = jax.nn.silu(jnp.dot(m1, e2w_ref[...], preferred_element_type=_f32)
                    + e2b_ref[...])
    cm1 = jax.nn.silu(jnp.dot(m, c1w_ref[...], preferred_element_type=_f32)
                      + c1b_ref[...])
    cm = jnp.dot(cm1, c2w_ref[...], preferred_element_type=_f32) + c2b_ref[...]
    wdiff = diff * cm
    zpad = jnp.zeros((m.shape[0], TW - H - 8), _f32)
    out_ref[...] = jnp.concatenate([m, wdiff, zpad], axis=1)


def _edge_mlp(g, eap, wc, wd, e1b, e2w, e2b, c1w, c1b, c2w, c2b):
    es = E_PAD // NSTRIPE
    grid = (es // TE,)
    return pl.pallas_call(
        _edge_body,
        grid=grid,
        in_specs=[pl.BlockSpec((TE, TW), lambda i: (i, 0)),
                  pl.BlockSpec((TE, 8), lambda i: (i, 0)),
                  _full((1, H)), _full((8, H)), _full((1, H)),
                  _full((H, H)), _full((1, H)),
                  _full((H, H)), _full((1, H)),
                  _full((H, 8)), _full((1, 8))],
        out_specs=pl.BlockSpec((TE, TW), lambda i: (i, 0)),
        out_shape=jax.ShapeDtypeStruct((E_PAD // NSTRIPE, TW), _f32),
    )(g, eap, wc, wd, e1b, e2w, e2b, c1w, c1b, c2w, c2b)


def _node_body(h_ref, cp_ref, p0_ref, p1_ref, p2_ref, p3_ref,
               n1h_ref, n1a_ref, n1b_ref,
               n2w_ref, n2b_ref, wa_ref, wb_ref,
               hn_ref, cpn_ref, t1_ref, t2_ref):
    h = h_ref[...]
    s = (p0_ref[...] + p1_ref[...]) + (p2_ref[...] + p3_ref[...])
    agg = s[:, :H]
    rows = pl.program_id(0) * TN + lax.broadcasted_iota(jnp.int32, (TN, 1), 0)
    valid = (rows < N).astype(_f32)
    cpn = (cp_ref[...] + s[:, H:H + 8]) * valid
    t = jnp.maximum(
        jnp.dot(h, n1h_ref[...], preferred_element_type=_f32)
        + jnp.dot(agg, n1a_ref[...], preferred_element_type=_f32)
        + n1b_ref[...], 0.0)
    hn = (h + jnp.dot(t, n2w_ref[...], preferred_element_type=_f32)
          + n2b_ref[...]) * valid
    ha = jnp.dot(hn, wa_ref[...], preferred_element_type=_f32)
    hb = jnp.dot(hn, wb_ref[...], preferred_element_type=_f32)
    zpad = jnp.zeros((cpn.shape[0], TW - H - 8), _f32)
    hn_ref[...] = hn
    cpn_ref[...] = cpn
    t1_ref[...] = jnp.concatenate([ha, cpn, zpad], axis=1)
    t2_ref[...] = jnp.concatenate([hb, -cpn, zpad], axis=1)


def _node_update(h, cp, p0, p1, p2, p3, n1h, n1a, n1b, n2w, n2b, wa, wb):
    grid = (N_PAD // TN,)
    return pl.pallas_call(
        _node_body,
        grid=grid,
        in_specs=[pl.BlockSpec((TN, H), lambda i: (i, 0)),
                  pl.BlockSpec((TN, 8), lambda i: (i, 0)),
                  pl.BlockSpec((TN, TW), lambda i: (i, 0)),
                  pl.BlockSpec((TN, TW), lambda i: (i, 0)),
                  pl.BlockSpec((TN, TW), lambda i: (i, 0)),
                  pl.BlockSpec((TN, TW), lambda i: (i, 0)),
                  _full((H, H)), _full((H, H)), _full((1, H)),
                  _full((H, H)), _full((1, H)),
                  _full((H, H)), _full((H, H))],
        out_specs=[pl.BlockSpec((TN, H), lambda i: (i, 0)),
                   pl.BlockSpec((TN, 8), lambda i: (i, 0)),
                   pl.BlockSpec((TN, TW), lambda i: (i, 0)),
                   pl.BlockSpec((TN, TW), lambda i: (i, 0))],
        out_shape=[jax.ShapeDtypeStruct((N_PAD, H), _f32),
                   jax.ShapeDtypeStruct((N_PAD, 8), _f32),
                   jax.ShapeDtypeStruct((N_PAD, TW), _f32),
                   jax.ShapeDtypeStruct((N_PAD, TW), _f32)],
    )(h, cp, p0, p1, p2, p3, n1h, n1a, n1b, n2w, n2b, wa, wb)


def _split_edge_weights(lp):
    e1 = lp["e1_w"]
    wa = e1[:H]
    wb = e1[H:2 * H]
    wc = e1[2 * H:2 * H + 1]
    wd = jnp.zeros((8, H), _f32).at[:4].set(e1[2 * H + 1:])
    c2w = jnp.zeros((H, 8), _f32).at[:, :3].set(lp["c2_w"])
    c2b = jnp.zeros((1, 8), _f32).at[0, :3].set(lp["c2_b"])
    return wa, wb, wc, wd, c2w, c2b


def kernel(node_features, edge_indices, edges_features, batch_size, params):
    row = edge_indices[0].astype(jnp.int32)
    col = edge_indices[1].astype(jnp.int32)
    pad = jnp.full((E_PAD - E,), N, jnp.int32)
    rowp = jnp.concatenate([row, pad])
    colp = jnp.concatenate([col, pad])
    rc = jnp.stack([rowp.reshape(-1, CHUNK), colp.reshape(-1, CHUNK)], axis=1)
    eap_full = jnp.pad(edges_features, ((0, E_PAD - E), (0, 4)))
    es = E_PAD // NSTRIPE
    eap = [eap_full[st * es:(st + 1) * es] for st in range(NSTRIPE)]
    nfp = jnp.pad(node_features, ((0, N_PAD - N), (0, 0)))
    zeros_tw = jnp.zeros((N_PAD, TW), _f32)

    ew = jnp.concatenate([jnp.zeros((3, H), _f32), params["emb_w"]])
    eb = params["emb_b"].reshape(1, H)

    layers = params["layers"]
    split = [_split_edge_weights(lp) for lp in layers]

    wa0, wb0 = split[0][0], split[0][1]
    h, cp, t1, t2 = _embed_tables(nfp, ew, eb, wa0, wb0)

    zh = jnp.zeros((H, H), _f32)
    for l in range(L):
        lp = layers[l]
        _, _, wc, wd, c2w, c2b = split[l]
        parts = []
        for st in range(NSTRIPE):
            g = _gather_edges(st, t1, t2, rc)
            mw = _edge_mlp(g, eap[st], wc, wd,
                           lp["e1_b"].reshape(1, H), lp["e2_w"],
                           lp["e2_b"].reshape(1, H), lp["c1_w"],
                           lp["c1_b"].reshape(1, H), c2w, c2b)
            parts.append(_scatter_edges(st, mw, rc, zeros_tw))
        wan, wbn = (split[l + 1][0], split[l + 1][1]) if l + 1 < L else (zh, zh)
        h, cp, t1, t2 = _node_update(
            h, cp, parts[0][:N_PAD], parts[0][N_PAD:],
            parts[1][:N_PAD], parts[1][N_PAD:],
            lp["n1_w"][:H], lp["n1_w"][H:], lp["n1_b"].reshape(1, H),
            lp["n2_w"], lp["n2_b"].reshape(1, H), wan, wbn)

    pred = cp[:N, :3]
    return (pred, edge_indices, edges_features)
